# baseline (speedup 1.0000x reference)
import functools

import jax
import jax.numpy as jnp
from jax import lax
from jax.experimental import pallas as pl
from jax.experimental.pallas import tpu as pltpu
from jax.experimental.pallas import tpu_sc as plsc

_LANES = 16
_NUM_CORES = 2
_NUM_SUBCORES = 16
_VOCAB_PAD = 1024
_NCHUNK = 4


def kernel(input_x, char_emb, weight_char_emb, fc1_w, fc1_b):
    B, M = input_x.shape
    V, E = char_emb.shape
    NW = _NUM_CORES * _NUM_SUBCORES
    BPW = B // NW
    CW = BPW // _NCHUNK
    GPC = CW // _LANES
    EPT = _VOCAB_PAD // _NUM_SUBCORES

    xt = input_x.T
    fcw = fc1_w.reshape(E)

    mesh = plsc.VectorSubcoreMesh(core_axis_name="c", subcore_axis_name="s")

    @functools.partial(
        pl.kernel,
        out_type=jax.ShapeDtypeStruct((B,), jnp.float32),
        mesh=mesh,
        compiler_params=pltpu.CompilerParams(needs_layout_passes=False),
        scratch_types=[
            pltpu.VMEM((2, M, CW), jnp.int32),
            pltpu.VMEM((_VOCAB_PAD,), jnp.float32),
            pltpu.VMEM((_VOCAB_PAD * _LANES,), jnp.float32),
            pltpu.VMEM((EPT, E), jnp.float32),
            pltpu.VMEM((E + _LANES,), jnp.float32),
            pltpu.VMEM((EPT,), jnp.float32),
            pltpu.VMEM((112,), jnp.float32),
            pltpu.VMEM((_LANES,), jnp.float32),
            pltpu.VMEM((BPW,), jnp.float32),
            pltpu.VMEM_SHARED((_VOCAB_PAD,), jnp.float32),
            pltpu.SemaphoreType.DMA,
            pltpu.SemaphoreType.DMA,
        ],
    )
    def sc_score(xt_hbm, emb_hbm, fcw_hbm, w_hbm, b_hbm, out_hbm,
                 x_v, v_v, vrep_v, embs_v, fcwe_v, vloc_v, w_v, b_v, o_v,
                 vsh, sem0, sem1):
        cid = lax.axis_index("c")
        sid = lax.axis_index("s")
        wid = sid * _NUM_CORES + cid
        base = wid * BPW
        iota = lax.iota(jnp.int32, _LANES)

        def start_chunk(c):
            return pltpu.async_copy(
                xt_hbm.at[:, pl.ds(base + c * CW, CW)],
                x_v.at[c & 1], sem0)

        start_chunk(0)

        ebase = jnp.minimum(sid * EPT, V - EPT)
        cps = [
            pltpu.async_copy(w_hbm, w_v.at[0:M], sem1),
            pltpu.async_copy(b_hbm, b_v.at[0:1], sem1),
            pltpu.async_copy(emb_hbm.at[pl.ds(ebase, EPT)], embs_v, sem1),
            pltpu.async_copy(fcw_hbm, fcwe_v.at[0:E], sem1),
            pltpu.async_copy(fcw_hbm.at[0:_LANES],
                             fcwe_v.at[pl.ds(E, _LANES)], sem1),
        ]
        for cp in cps:
            cp.wait()

        for g2 in range(EPT // _LANES):
            rowv = g2 * _LANES + iota

            def e_body(e, acc):
                col = (iota + e) & (E - 1)
                wrot = fcwe_v[pl.ds(e, _LANES)]
                ev = plsc.load_gather(embs_v, [rowv, col])
                return acc + ev * wrot

            acc = lax.fori_loop(
                0, E, e_body, jnp.zeros((_LANES,), jnp.float32), unroll=4)
            vloc_v[pl.ds(g2 * _LANES, _LANES)] = acc

        pltpu.sync_copy(vloc_v, vsh.at[pl.ds(ebase, EPT)])
        plsc.subcore_barrier()
        pltpu.sync_copy(vsh, v_v)

        def rep_body(jc, carry):
            chunk = v_v[pl.ds(jc * _LANES, _LANES)]
            for i in range(_LANES):
                vrep_v[pl.ds((jc * _LANES + i) * _LANES, _LANES)] = (
                    jnp.full((_LANES,), chunk[i], jnp.float32))
            return carry

        lax.fori_loop(0, (V + _LANES - 1) // _LANES, rep_body, 0)

        bias = b_v[pl.ds(0, _LANES)][0]
        MFULL = M // _LANES
        MTAIL = M % _LANES
        w_tail = w_v[pl.ds(MFULL * _LANES, _LANES)]

        def g_body(g, carry):
            c = g // GPC
            gc = g % GPC

            @pl.when(jnp.logical_and(gc == 0, c + 1 < _NCHUNK))
            def _():
                start_chunk(c + 1)

            @pl.when(gc == 0)
            def _():
                pltpu.make_async_copy(
                    xt_hbm.at[:, pl.ds(base, CW)], x_v.at[0], sem0).wait()

            xc = x_v.at[c & 1]
            gb = gc * _LANES

            def mc_body(mc, acc):
                wc = w_v[pl.ds(mc * _LANES, _LANES)]
                mb = mc * _LANES
                for i in range(_LANES):
                    idx = xc[mb + i, pl.ds(gb, _LANES)]
                    gv = plsc.load_gather(vrep_v, [(idx << 4) + iota])
                    acc = acc + gv * wc[i]
                return acc

            acc = lax.fori_loop(
                0, MFULL, mc_body, jnp.zeros((_LANES,), jnp.float32))
            for i in range(MTAIL):
                idx = xc[MFULL * _LANES + i, pl.ds(gb, _LANES)]
                gv = plsc.load_gather(vrep_v, [(idx << 4) + iota])
                acc = acc + gv * w_tail[i]
            o_v[pl.ds(g * _LANES, _LANES)] = acc + bias
            return carry

        lax.fori_loop(0, _NCHUNK * GPC, g_body, 0)

        pltpu.sync_copy(o_v, out_hbm.at[pl.ds(base, BPW)])

    return sc_score(xt, char_emb, fcw, weight_char_emb, fc1_b)


# speedup vs baseline: 1.0126x; 1.0126x over previous
import functools

import jax
import jax.numpy as jnp
from jax import lax
from jax.experimental import pallas as pl
from jax.experimental.pallas import tpu as pltpu
from jax.experimental.pallas import tpu_sc as plsc

_LANES = 16
_NUM_CORES = 2
_NUM_SUBCORES = 16
_VOCAB_PAD = 1024
_NCHUNK = 4


def kernel(input_x, char_emb, weight_char_emb, fc1_w, fc1_b):
    B, M = input_x.shape
    V, E = char_emb.shape
    NW = _NUM_CORES * _NUM_SUBCORES
    BPW = B // NW
    CW = BPW // _NCHUNK
    GPC = CW // _LANES
    EPT = 128

    xt = input_x.T
    embT = char_emb.T
    fcw = fc1_w.reshape(E)

    mesh = plsc.VectorSubcoreMesh(core_axis_name="c", subcore_axis_name="s")

    @functools.partial(
        pl.kernel,
        out_type=jax.ShapeDtypeStruct((B,), jnp.float32),
        mesh=mesh,
        compiler_params=pltpu.CompilerParams(needs_layout_passes=False),
        scratch_types=[
            pltpu.VMEM((2, M, CW), jnp.int32),
            pltpu.VMEM((_VOCAB_PAD,), jnp.float32),
            pltpu.VMEM((_VOCAB_PAD * _LANES,), jnp.float32),
            pltpu.VMEM((E, EPT), jnp.float32),
            pltpu.VMEM((E + _LANES,), jnp.float32),
            pltpu.VMEM((EPT,), jnp.float32),
            pltpu.VMEM((112,), jnp.float32),
            pltpu.VMEM((_LANES,), jnp.float32),
            pltpu.VMEM((BPW,), jnp.float32),
            pltpu.VMEM_SHARED((_VOCAB_PAD,), jnp.float32),
            pltpu.SemaphoreType.DMA,
            pltpu.SemaphoreType.DMA,
        ],
    )
    def sc_score(xt_hbm, embT_hbm, fcw_hbm, w_hbm, b_hbm, out_hbm,
                 x_v, v_v, vrep_v, embs_v, fcwe_v, vloc_v, w_v, b_v, o_v,
                 vsh, sem0, sem1):
        cid = lax.axis_index("c")
        sid = lax.axis_index("s")
        wid = sid * _NUM_CORES + cid
        base = wid * BPW
        iota = lax.iota(jnp.int32, _LANES)

        def start_chunk(c):
            return pltpu.async_copy(
                xt_hbm.at[:, pl.ds(base + c * CW, CW)],
                x_v.at[c & 1], sem0)

        start_chunk(0)

        ebase = (sid & 7) * EPT
        cps = [
            pltpu.async_copy(w_hbm, w_v.at[0:M], sem1),
            pltpu.async_copy(b_hbm, b_v.at[0:1], sem1),
            pltpu.async_copy(embT_hbm.at[:, pl.ds(ebase, EPT)], embs_v,
                             sem1),
            pltpu.async_copy(fcw_hbm, fcwe_v.at[0:E], sem1),
        ]
        for cp in cps:
            cp.wait()

        for g2 in range(EPT // _LANES):
            rb = g2 * _LANES

            def e_body(e, acc):
                w_e = fcwe_v[pl.ds(e, _LANES)][0]
                ev = embs_v[e, pl.ds(rb, _LANES)]
                return acc + ev * w_e

            acc = lax.fori_loop(
                0, E, e_body, jnp.zeros((_LANES,), jnp.float32), unroll=4)
            vloc_v[pl.ds(g2 * _LANES, _LANES)] = acc

        @pl.when(sid < 8)
        def _():
            pltpu.sync_copy(vloc_v, vsh.at[pl.ds(ebase, EPT)])

        plsc.subcore_barrier()
        pltpu.sync_copy(vsh, v_v)

        def rep_body(jc, carry):
            chunk = v_v[pl.ds(jc * _LANES, _LANES)]
            for i in range(_LANES):
                vrep_v[pl.ds((jc * _LANES + i) * _LANES, _LANES)] = (
                    jnp.full((_LANES,), chunk[i], jnp.float32))
            return carry

        lax.fori_loop(0, (V + _LANES - 1) // _LANES, rep_body, 0)

        bias = b_v[pl.ds(0, _LANES)][0]
        MFULL = M // _LANES
        MTAIL = M % _LANES
        w_tail = w_v[pl.ds(MFULL * _LANES, _LANES)]

        def g_body(g, carry):
            c = g // GPC
            gc = g % GPC

            @pl.when(jnp.logical_and(gc == 0, c + 1 < _NCHUNK))
            def _():
                start_chunk(c + 1)

            @pl.when(gc == 0)
            def _():
                pltpu.make_async_copy(
                    xt_hbm.at[:, pl.ds(base, CW)], x_v.at[0], sem0).wait()

            xc = x_v.at[c & 1]
            gb = gc * _LANES

            def mc_body(mc, acc):
                wc = w_v[pl.ds(mc * _LANES, _LANES)]
                mb = mc * _LANES
                for i in range(_LANES):
                    idx = xc[mb + i, pl.ds(gb, _LANES)]
                    gv = plsc.load_gather(vrep_v, [(idx << 4) + iota])
                    acc = acc + gv * wc[i]
                return acc

            acc = lax.fori_loop(
                0, MFULL, mc_body, jnp.zeros((_LANES,), jnp.float32))
            for i in range(MTAIL):
                idx = xc[MFULL * _LANES + i, pl.ds(gb, _LANES)]
                gv = plsc.load_gather(vrep_v, [(idx << 4) + iota])
                acc = acc + gv * w_tail[i]
            o_v[pl.ds(g * _LANES, _LANES)] = acc + bias
            return carry

        lax.fori_loop(0, _NCHUNK * GPC, g_body, 0)

        pltpu.sync_copy(o_v, out_hbm.at[pl.ds(base, BPW)])

    return sc_score(xt, embT, fcw, weight_char_emb, fc1_b)
